# baseline (speedup 1.0000x reference)
import functools

import jax
import jax.numpy as jnp
from jax import lax
from jax.experimental import pallas as pl
from jax.experimental.pallas import tpu as pltpu
from jax.experimental.pallas import tpu_sc as plsc

B = 1024
S = 200
D = 64
N = B * S
NC = 2
NS = 16
NW = NC * NS
CHUNK = 128
PER_W = N // NW
CHUNKS_W = PER_W // CHUNK
CPS = B // CHUNK
V2 = 500000

_mesh = plsc.VectorSubcoreMesh(core_axis_name="c", subcore_axis_name="s")


@functools.partial(
    pl.kernel,
    mesh=_mesh,
    compiler_params=pltpu.CompilerParams(
        use_tc_tiling_on_sc=True, needs_layout_passes=False),
    out_type=jax.ShapeDtypeStruct((S, D, B), jnp.float32),
    scratch_types=[
        pltpu.VMEM((CHUNKS_W, CHUNK), jnp.int32),
        pltpu.VMEM((CHUNKS_W, CHUNK), jnp.int32),
        pltpu.VMEM((2, CHUNK, 2 * D), jnp.float32),
        pltpu.VMEM((2, D, CHUNK), jnp.float32),
        pltpu.SemaphoreType.DMA,
        pltpu.SemaphoreType.DMA,
        pltpu.SemaphoreType.DMA,
        pltpu.SemaphoreType.DMA,
    ],
)
def _gather(idx_hbm, table_hbm, out_hbm, idx_v, idx2_v, bufs, packs,
            g0, g1, w0, w1):
    wid = lax.axis_index("s") * NC + lax.axis_index("c")
    chunk0 = wid * CHUNKS_W
    gsem = (g0, g1)
    wsem = (w0, w1)
    lanes = lax.iota(jnp.int32, 16)

    pltpu.sync_copy(idx_hbm.at[wid], idx_v)

    def shift_body(j, carry):
        for g in range(CHUNK // 16):
            v = idx_v[j, pl.ds(g * 16, 16)]
            idx2_v[j, pl.ds(g * 16, 16)] = lax.shift_right_logical(v, 1)
        return carry

    lax.fori_loop(0, CHUNKS_W, shift_body, 0)

    def start_gather(j, b):
        pltpu.async_copy(table_hbm.at[idx2_v.at[j]], bufs.at[b], gsem[b])

    def drain_gather(j, b):
        pltpu.make_async_copy(
            table_hbm.at[idx2_v.at[j]], bufs.at[b], gsem[b]).wait()

    def compact(j, b):
        buf = bufs.at[b]
        for g in range(CHUNK // 16):
            rows = g * 16 + lanes
            half = lax.mul(
                lax.bitwise_and(idx_v[j, pl.ds(g * 16, 16)], 1), D)
            for d in range(D):
                vals = plsc.load_gather(buf, [rows, half + d])
                packs[b, d, pl.ds(g * 16, 16)] = vals

    def writeback(j, b):
        c_g = chunk0 + j
        s_i = lax.div(c_g, CPS)
        b0 = lax.mul(lax.rem(c_g, CPS), CHUNK)
        return pltpu.make_async_copy(
            packs.at[b], out_hbm.at[s_i, :, pl.ds(b0, CHUNK)], wsem[b])

    start_gather(0, 0)
    start_gather(1, 1)

    def body(jj, carry):
        for b in range(2):
            j = 2 * jj + b
            drain_gather(j, b)

            @pl.when(j >= 2)
            def _():
                writeback(j - 2, b).wait()

            compact(j, b)

            @pl.when(j + 2 < CHUNKS_W)
            def _():
                start_gather(j + 2, b)

            writeback(j, b).start()
        return carry

    lax.fori_loop(0, CHUNKS_W // 2, body, 0)
    writeback(CHUNKS_W - 2, 0).wait()
    writeback(CHUNKS_W - 1, 1).wait()


def _mask_body(x_ref, o_ref):
    o_ref[...] = (x_ref[...] != 0).astype(jnp.float32)


_mask_call = pl.pallas_call(
    _mask_body,
    out_shape=jax.ShapeDtypeStruct((B, S), jnp.float32),
)


def kernel(x, weight):
    xt = jnp.transpose(x).reshape(NW, CHUNKS_W, CHUNK).astype(jnp.int32)
    w2 = jnp.reshape(weight, (V2, 2 * D))
    out = _gather(xt, w2)
    mask = _mask_call(x)
    return jnp.transpose(out, (0, 2, 1)), mask
